# baseline (speedup 1.0000x reference)
import functools

import jax
import jax.numpy as jnp
from jax import lax
from jax.experimental import pallas as pl
from jax.experimental.pallas import tpu as pltpu
from jax.experimental.pallas import tpu_sc as plsc

D = H = W = 128
N = 262144
NPTS = 2 * N
NC, NS, L = 2, 16, 16
PT = NPTS // NS
CH = 2048
NCHUNK = PT // CH
VPC = CH // L
ROWS = 4 * CH // 128
HALF = (D // 2) * H * W
SHARE = HALF // NS
HCH = 8192


def _body(rx, ry, rz, cx, cy, cz, out_hbm,
          pbuf, idx_buf, val_buf, zbuf, grid, acc_buf, scsem, szsem, shsem):
    cid = lax.axis_index("c")
    sid = lax.axis_index("s")
    parity = cid

    base_pt = sid * PT
    cbase = base_pt - jnp.where(base_pt >= N, N, 0)

    def zloop(k, _):
        zbuf[0, pl.ds(k * L, L)] = jnp.zeros((L,), jnp.float32)
        return 0
    lax.fori_loop(0, HCH // L, zloop, 0)
    zdescs = [
        pltpu.async_copy(zbuf.at[0], grid.at[pl.ds(sid * SHARE + k * HCH, HCH)],
                         szsem)
        for k in range(SHARE // HCH)
    ]
    for d in zdescs:
        d.wait()
    plsc.subcore_barrier()

    sign = jnp.where(base_pt < N, 1.0, -1.0).astype(jnp.float32)

    def chunk_loop(c, _):
        off = base_pt + c * CH
        coff = cbase + c * CH
        pltpu.sync_copy(rx.at[pl.ds(off, CH)], pbuf.at[0])
        pltpu.sync_copy(ry.at[pl.ds(off, CH)], pbuf.at[1])
        pltpu.sync_copy(rz.at[pl.ds(off, CH)], pbuf.at[2])
        pltpu.sync_copy(cx.at[pl.ds(coff, CH)], pbuf.at[3])
        pltpu.sync_copy(cy.at[pl.ds(coff, CH)], pbuf.at[4])
        pltpu.sync_copy(cz.at[pl.ds(coff, CH)], pbuf.at[5])

        def vec_loop(j, _):
            s = j * L
            x = pbuf[0, pl.ds(s, L)] + pbuf[3, pl.ds(s, L)]
            y = pbuf[1, pl.ds(s, L)] + pbuf[4, pl.ds(s, L)]
            z = pbuf[2, pl.ds(s, L)] + pbuf[5, pl.ds(s, L)]
            xg = x * 64.0 + 63.5
            yg = y * 64.0 + 63.5
            zg = z * 64.0 + 63.5
            x0 = (xg + 1024.0).astype(jnp.int32) - 1024
            y0 = (yg + 1024.0).astype(jnp.int32) - 1024
            z0 = (zg + 1024.0).astype(jnp.int32) - 1024
            fx = xg - x0.astype(jnp.float32)
            fy = yg - y0.astype(jnp.float32)
            fz = zg - z0.astype(jnp.float32)
            wx0 = jnp.where(x0.astype(jnp.uint32) <= 127, 1.0 - fx, 0.0)
            wx1 = jnp.where((x0 + 1).astype(jnp.uint32) <= 127, fx, 0.0)
            wy0 = jnp.where(y0.astype(jnp.uint32) <= 127, 1.0 - fy, 0.0)
            wy1 = jnp.where((y0 + 1).astype(jnp.uint32) <= 127, fy, 0.0)
            xc0 = x0 & 127
            xc1 = (x0 + 1) & 127
            yb0 = (y0 & 127) * 128
            yb1 = ((y0 + 1) & 127) * 128
            dlt = (z0 ^ parity) & 1
            zci = z0 + dlt
            wz = jnp.where(dlt == 0, 1.0 - fz, fz)
            wz = jnp.where(zci.astype(jnp.uint32) <= 127, wz, 0.0)
            wz = wz * sign
            zb = ((zci & 127) >> 1) * (H * W)
            a0 = wz * wy0
            a1 = wz * wy1
            b0 = zb + yb0
            b1 = zb + yb1
            row = j >> 1
            col = (j & 1) * 64
            idx_buf[row, pl.ds(col, L)] = b0 + xc0
            val_buf[row, pl.ds(col, L)] = a0 * wx0
            idx_buf[row, pl.ds(col + 16, L)] = b0 + xc1
            val_buf[row, pl.ds(col + 16, L)] = a0 * wx1
            idx_buf[row, pl.ds(col + 32, L)] = b1 + xc0
            val_buf[row, pl.ds(col + 32, L)] = a1 * wx0
            idx_buf[row, pl.ds(col + 48, L)] = b1 + xc1
            val_buf[row, pl.ds(col + 48, L)] = a1 * wx1
            return 0
        prev = None
        for g in range(16):
            lax.fori_loop(g * 8, g * 8 + 8, vec_loop, 0)
            fired = [
                pltpu.async_copy(val_buf.at[g * 4 + u], grid.at[idx_buf.at[g * 4 + u]],
                                 scsem, add=True)
                for u in range(4)
            ]
            if prev:
                for d in prev:
                    d.wait()
            prev = fired
        for d in prev:
            d.wait()
        return 0
    lax.fori_loop(0, NCHUNK, chunk_loop, 0)

    plsc.subcore_barrier()

    HN = SHARE // HCH
    gbase = sid * SHARE
    acc = jnp.zeros((L,), jnp.float32)
    d0 = pltpu.async_copy(grid.at[pl.ds(gbase, HCH)], zbuf.at[0], shsem)
    pend = d0
    for k in range(HN):
        pend.wait()
        if k + 1 < HN:
            pend = pltpu.async_copy(
                grid.at[pl.ds(gbase + (k + 1) * HCH, HCH)],
                zbuf.at[(k + 1) & 1], shsem)
        def hvec(i, acc, _b=k & 1):
            d = zbuf[_b, pl.ds(i * L, L)]
            ad = jnp.abs(d)
            m = jnp.minimum(ad, 1.0)
            return acc + m * (ad - 0.5 * m)
        acc = lax.fori_loop(0, HCH // L, hvec, acc)
    acc_buf[...] = acc
    pltpu.sync_copy(acc_buf, out_hbm.at[cid * NS + sid])


@jax.jit
def _splat_loss(rx, ry, rz, cx, cy, cz):
    mesh = plsc.VectorSubcoreMesh(core_axis_name="c", subcore_axis_name="s")
    fn = pl.kernel(
        _body,
        out_type=jax.ShapeDtypeStruct((NC * NS, L), jnp.float32),
        mesh=mesh,
        scratch_types=[
            pltpu.VMEM((6, CH), jnp.float32),
            pltpu.VMEM((ROWS, 128), jnp.int32),
            pltpu.VMEM((ROWS, 128), jnp.float32),
            pltpu.VMEM((2, HCH), jnp.float32),
            pltpu.VMEM_SHARED((HALF,), jnp.float32),
            pltpu.VMEM((L,), jnp.float32),
            pltpu.SemaphoreType.DMA,
            pltpu.SemaphoreType.DMA,
            pltpu.SemaphoreType.DMA,
        ],
    )
    return fn(rx, ry, rz, cx, cy, cz)


def kernel(registration_pred, registration_gt, coords):
    p = registration_pred[0]
    g = registration_gt[0]
    c = coords[0]
    parts = _splat_loss(
        jnp.concatenate([p[:, 0], g[:, 0]]),
        jnp.concatenate([p[:, 1], g[:, 1]]),
        jnp.concatenate([p[:, 2], g[:, 2]]),
        c[:, 0], c[:, 1], c[:, 2])
    return jnp.sum(parts)


# speedup vs baseline: 1.2976x; 1.2976x over previous
import functools

import jax
import jax.numpy as jnp
from jax import lax
from jax.experimental import pallas as pl
from jax.experimental.pallas import tpu as pltpu
from jax.experimental.pallas import tpu_sc as plsc

D = H = W = 128
N = 262144
NPTS = 2 * N
NC, NS, L = 2, 16, 16
PT = NPTS // NS
CH = 2048
NCHUNK = PT // CH
VPC = CH // L
ROWS = 4 * CH // 128
HALF = (D // 2) * H * W
SHARE = HALF // NS
HCH = 8192


def _body(rx, ry, rz, cx, cy, cz, out_hbm,
          pbuf, idx_buf, val_buf, zbuf, grid, acc_buf, scsem, szsem, shsem,
          sgsem):
    cid = lax.axis_index("c")
    sid = lax.axis_index("s")
    parity = cid

    base_pt = sid * PT
    cbase = base_pt - jnp.where(base_pt >= N, N, 0)

    def zloop(k, _):
        zbuf[0, pl.ds(k * L, L)] = jnp.zeros((L,), jnp.float32)
        return 0
    lax.fori_loop(0, HCH // L, zloop, 0)
    zdescs = [
        pltpu.async_copy(zbuf.at[0], grid.at[pl.ds(sid * SHARE + k * HCH, HCH)],
                         szsem)
        for k in range(SHARE // HCH)
    ]
    for d in zdescs:
        d.wait()
    plsc.subcore_barrier()

    sign = jnp.where(base_pt < N, 1.0, -1.0).astype(jnp.float32)

    def chunk_loop(c, _):
        off = base_pt + c * CH
        coff = cbase + c * CH
        sdescs = [
            pltpu.async_copy(rx.at[pl.ds(off, CH)], pbuf.at[0], sgsem),
            pltpu.async_copy(ry.at[pl.ds(off, CH)], pbuf.at[1], sgsem),
            pltpu.async_copy(rz.at[pl.ds(off, CH)], pbuf.at[2], sgsem),
            pltpu.async_copy(cx.at[pl.ds(coff, CH)], pbuf.at[3], sgsem),
            pltpu.async_copy(cy.at[pl.ds(coff, CH)], pbuf.at[4], sgsem),
            pltpu.async_copy(cz.at[pl.ds(coff, CH)], pbuf.at[5], sgsem),
        ]
        for d in sdescs:
            d.wait()

        def vec_loop(j, _):
            s = j * L
            x = pbuf[0, pl.ds(s, L)] + pbuf[3, pl.ds(s, L)]
            y = pbuf[1, pl.ds(s, L)] + pbuf[4, pl.ds(s, L)]
            z = pbuf[2, pl.ds(s, L)] + pbuf[5, pl.ds(s, L)]
            xg = x * 64.0 + 63.5
            yg = y * 64.0 + 63.5
            zg = z * 64.0 + 63.5
            x0 = (xg + 1024.0).astype(jnp.int32) - 1024
            y0 = (yg + 1024.0).astype(jnp.int32) - 1024
            z0 = (zg + 1024.0).astype(jnp.int32) - 1024
            fx = xg - x0.astype(jnp.float32)
            fy = yg - y0.astype(jnp.float32)
            fz = zg - z0.astype(jnp.float32)
            wx0 = jnp.where(x0.astype(jnp.uint32) <= 127, 1.0 - fx, 0.0)
            wx1 = jnp.where((x0 + 1).astype(jnp.uint32) <= 127, fx, 0.0)
            wy0 = jnp.where(y0.astype(jnp.uint32) <= 127, 1.0 - fy, 0.0)
            wy1 = jnp.where((y0 + 1).astype(jnp.uint32) <= 127, fy, 0.0)
            xc0 = x0 & 127
            xc1 = (x0 + 1) & 127
            yb0 = (y0 & 127) * 128
            yb1 = ((y0 + 1) & 127) * 128
            dlt = (z0 ^ parity) & 1
            zci = z0 + dlt
            wz = jnp.where(dlt == 0, 1.0 - fz, fz)
            wz = jnp.where(zci.astype(jnp.uint32) <= 127, wz, 0.0)
            wz = wz * sign
            zb = ((zci & 127) >> 1) * (H * W)
            a0 = wz * wy0
            a1 = wz * wy1
            b0 = zb + yb0
            b1 = zb + yb1
            row = j >> 1
            col = (j & 1) * 64
            idx_buf[row, pl.ds(col, L)] = b0 + xc0
            val_buf[row, pl.ds(col, L)] = a0 * wx0
            idx_buf[row, pl.ds(col + 16, L)] = b0 + xc1
            val_buf[row, pl.ds(col + 16, L)] = a0 * wx1
            idx_buf[row, pl.ds(col + 32, L)] = b1 + xc0
            val_buf[row, pl.ds(col + 32, L)] = a1 * wx0
            idx_buf[row, pl.ds(col + 48, L)] = b1 + xc1
            val_buf[row, pl.ds(col + 48, L)] = a1 * wx1
            return 0
        prev = None
        for g in range(16):
            lax.fori_loop(g * 8, g * 8 + 8, vec_loop, 0)
            fired = [
                pltpu.async_copy(val_buf.at[g * 4 + u], grid.at[idx_buf.at[g * 4 + u]],
                                 scsem, add=True)
                for u in range(4)
            ]
            if prev:
                for d in prev:
                    d.wait()
            prev = fired
        for d in prev:
            d.wait()
        return 0
    lax.fori_loop(0, NCHUNK, chunk_loop, 0)

    plsc.subcore_barrier()

    HN = SHARE // HCH
    gbase = sid * SHARE
    acc = jnp.zeros((L,), jnp.float32)
    d0 = pltpu.async_copy(grid.at[pl.ds(gbase, HCH)], zbuf.at[0], shsem)
    pend = d0
    for k in range(HN):
        pend.wait()
        if k + 1 < HN:
            pend = pltpu.async_copy(
                grid.at[pl.ds(gbase + (k + 1) * HCH, HCH)],
                zbuf.at[(k + 1) & 1], shsem)
        def hvec(i, acc, _b=k & 1):
            d = zbuf[_b, pl.ds(i * L, L)]
            ad = jnp.abs(d)
            m = jnp.minimum(ad, 1.0)
            return acc + m * (ad - 0.5 * m)
        acc = lax.fori_loop(0, HCH // L, hvec, acc)
    acc_buf[...] = acc
    pltpu.sync_copy(acc_buf, out_hbm.at[cid * NS + sid])


@jax.jit
def _splat_loss(rx, ry, rz, cx, cy, cz):
    mesh = plsc.VectorSubcoreMesh(core_axis_name="c", subcore_axis_name="s")
    fn = pl.kernel(
        _body,
        out_type=jax.ShapeDtypeStruct((NC * NS, L), jnp.float32),
        mesh=mesh,
        scratch_types=[
            pltpu.VMEM((6, CH), jnp.float32),
            pltpu.VMEM((ROWS, 128), jnp.int32),
            pltpu.VMEM((ROWS, 128), jnp.float32),
            pltpu.VMEM((2, HCH), jnp.float32),
            pltpu.VMEM_SHARED((HALF,), jnp.float32),
            pltpu.VMEM((L,), jnp.float32),
            pltpu.SemaphoreType.DMA,
            pltpu.SemaphoreType.DMA,
            pltpu.SemaphoreType.DMA,
            pltpu.SemaphoreType.DMA,
        ],
    )
    return fn(rx, ry, rz, cx, cy, cz)


def kernel(registration_pred, registration_gt, coords):
    p = registration_pred[0]
    g = registration_gt[0]
    c = coords[0]
    parts = _splat_loss(
        jnp.concatenate([p[:, 0], g[:, 0]]),
        jnp.concatenate([p[:, 1], g[:, 1]]),
        jnp.concatenate([p[:, 2], g[:, 2]]),
        c[:, 0], c[:, 1], c[:, 2])
    return jnp.sum(parts)
